# baseline (speedup 1.0000x reference)
import functools

import jax
import jax.numpy as jnp
from jax import lax
from jax.experimental import pallas as pl
from jax.experimental.pallas import tpu as pltpu
from jax.experimental.pallas import tpu_sc as plsc

NC, NS = 2, 16
NW = NC * NS
HIST = 200
BATCH = 4096
EMB = 64
OUT = 46
BPW = BATCH // NW
LANES = EMB // 16


def _emb_bag_sc(seq, emb_table):
    mesh = plsc.VectorSubcoreMesh(core_axis_name="c", subcore_axis_name="s")

    @functools.partial(
        pl.kernel,
        out_type=jax.ShapeDtypeStruct((BATCH, EMB), jnp.float32),
        mesh=mesh,
        compiler_params=pltpu.CompilerParams(use_tc_tiling_on_sc=False),
        scratch_types=[
            pltpu.VMEM((HIST, BPW), jnp.int32),
            pltpu.VMEM((BPW, EMB), jnp.float32),
            pltpu.VMEM((BPW, EMB), jnp.float32),
            pltpu.VMEM((BPW, EMB), jnp.float32),
            pltpu.SemaphoreType.DMA,
            pltpu.SemaphoreType.DMA,
        ],
    )
    def bag(seq_hbm, table_hbm, out_hbm, idx_v, rows0, rows1, acc, sem0, sem1):
        wid = lax.axis_index("s") * NC + lax.axis_index("c")
        base = wid * BPW
        pltpu.sync_copy(seq_hbm.at[:, pl.ds(base, BPW)], idx_v)

        pltpu.async_copy(table_hbm.at[idx_v.at[0]], rows0, sem0)
        zeros = jnp.zeros((16,), jnp.float32)

        @pl.loop(0, BPW)
        def _(j):
            for k in range(LANES):
                acc[j, pl.ds(k * 16, 16)] = zeros

        def accumulate(rows):
            @pl.loop(0, BPW, unroll=4)
            def _(j):
                for k in range(LANES):
                    sl = pl.ds(k * 16, 16)
                    plsc.addupdate(acc.at[j, sl], rows[j, sl])

        @pl.loop(0, HIST - 2, step=2)
        def _(h):
            pltpu.make_async_copy(table_hbm.at[idx_v.at[h]], rows0, sem0).wait()
            d1 = pltpu.async_copy(table_hbm.at[idx_v.at[h + 1]], rows1, sem1)
            accumulate(rows0)
            d1.wait()
            pltpu.async_copy(table_hbm.at[idx_v.at[h + 2]], rows0, sem0)
            accumulate(rows1)

        pltpu.make_async_copy(
            table_hbm.at[idx_v.at[HIST - 2]], rows0, sem0).wait()
        dl = pltpu.async_copy(table_hbm.at[idx_v.at[HIST - 1]], rows1, sem1)
        accumulate(rows0)
        dl.wait()
        accumulate(rows1)

        pltpu.sync_copy(acc, out_hbm.at[pl.ds(base, BPW)])

    return bag(seq, emb_table)


def _linear_tc(emb, W, b2):
    def mm(x_ref, w_ref, b_ref, o_ref):
        o_ref[...] = lax.dot_general(
            x_ref[...], w_ref[...],
            (((1,), (1,)), ((), ())),
            preferred_element_type=jnp.float32,
        ) + b_ref[...]

    return pl.pallas_call(
        mm,
        out_shape=jax.ShapeDtypeStruct((BATCH, OUT), jnp.float32),
    )(emb, W, b2)


def kernel(seq, emb_table, W, b):
    emb = _emb_bag_sc(seq.astype(jnp.int32), emb_table)
    return _linear_tc(emb, W, b.reshape(1, OUT))
